# baseline (speedup 1.0000x reference)
import functools

import jax
import jax.numpy as jnp
from jax import lax
from jax.experimental import pallas as pl
from jax.experimental.pallas import tpu as pltpu
from jax.experimental.pallas import tpu_sc as plsc

E = 8
D = 1024
R = 128
N = 4096
RT = 512
NCH = N // RT
GT = 256
NW = 32
PW = N // NW
SCCH = 4
SCB = PW // SCCH


def _argmin_pid(xt, c):
    xn = jnp.sum(xt * xt, axis=1, keepdims=True)
    dots = lax.dot_general(xt, c, (((1,), (1,)), ((), ())),
                           preferred_element_type=jnp.float32)
    cn = jnp.sum(c * c, axis=1)
    d2 = xn - 2.0 * dots + cn[None, :]
    bestv = d2[:, 0:1]
    bestid = jnp.zeros((xt.shape[0], 1), dtype=jnp.int32)
    for e in range(1, E):
        v = d2[:, e:e + 1]
        take = v < bestv
        bestid = jnp.where(take, e, bestid)
        bestv = jnp.where(take, v, bestv)
    return bestid


def _route_body(x_ref, c_ref, dest_ref, offs_ref, c_s):
    i = pl.program_id(0)
    bestid = _argmin_pid(x_ref[...], c_ref[...])
    onehot = (bestid == lax.broadcasted_iota(jnp.int32, (1, E), 1))
    c_s[pl.ds(i * RT, RT), :] = onehot.astype(jnp.float32)

    @pl.when(i == NCH - 1)
    def _finalize():
        ccat = jnp.concatenate(
            [c_s[ch * RT:(ch + 1) * RT, :] for ch in range(NCH)], axis=1)
        r0 = lax.broadcasted_iota(jnp.int32, (RT, RT), 0)
        r1 = lax.broadcasted_iota(jnp.int32, (RT, RT), 1)
        tlow = jnp.where(r0 >= r1, 1.0, 0.0)
        incl = lax.dot_general(tlow, ccat, (((1,), (0,)), ((), ())),
                               preferred_element_type=jnp.float32,
                               precision=lax.Precision.HIGHEST)
        tot = incl[RT - 1:RT, :]
        li = lax.broadcasted_iota(jnp.int32, (NCH * E, NCH * E), 0)
        lj = lax.broadcasted_iota(jnp.int32, (NCH * E, NCH * E), 1)
        tch = jnp.where((li % E == lj % E) & (li // E < lj // E), 1.0, 0.0)
        prior = lax.dot_general(tot, tch, (((1,), (0,)), ((), ())),
                                preferred_element_type=jnp.float32,
                               precision=lax.Precision.HIGHEST)
        gincl = incl + prior
        si = lax.broadcasted_iota(jnp.int32, (NCH * E, E), 0)
        sj = lax.broadcasted_iota(jnp.int32, (NCH * E, E), 1)
        counts = lax.dot_general(
            tot, jnp.where(si % E == sj, 1.0, 0.0), (((1,), (0,)), ((), ())),
            preferred_element_type=jnp.float32,
                               precision=lax.Precision.HIGHEST)
        bi = lax.broadcasted_iota(jnp.int32, (E, NCH * E), 0)
        bj = lax.broadcasted_iota(jnp.int32, (E, NCH * E), 1)
        starts64 = lax.dot_general(
            counts, jnp.where(bi < bj % E, 1.0, 0.0), (((1,), (0,)), ((), ())),
            preferred_element_type=jnp.float32,
                               precision=lax.Precision.HIGHEST)
        d64 = ccat * (starts64 + gincl - 1.0)
        gi = lax.broadcasted_iota(jnp.int32, (NCH * E, 128), 0)
        gj = lax.broadcasted_iota(jnp.int32, (NCH * E, 128), 1)
        dest = lax.dot_general(
            d64, jnp.where(gi // E == gj, 1.0, 0.0), (((1,), (0,)), ((), ())),
            preferred_element_type=jnp.float32,
                               precision=lax.Precision.HIGHEST)
        dest_ref[...] = dest.astype(jnp.int32)
        oi = lax.broadcasted_iota(jnp.int32, (E, 128), 0)
        oj = lax.broadcasted_iota(jnp.int32, (E, 128), 1)
        omat = jnp.where((oj < E) & (oi < oj % E), 1.0, 0.0) + \
            jnp.where((oj >= E) & (oj < 2 * E) & (oi <= oj % E), 1.0, 0.0)
        offs = lax.dot_general(counts, omat, (((1,), (0,)), ((), ())),
                               preferred_element_type=jnp.float32,
                               precision=lax.Precision.HIGHEST)
        offs_ref[...] = offs.astype(jnp.int32)


def _route(xf, centroids):
    return pl.pallas_call(
        _route_body,
        grid=(NCH,),
        in_specs=[
            pl.BlockSpec((RT, D), lambda i: (i, 0)),
            pl.BlockSpec((E, D), lambda i: (0, 0)),
        ],
        out_specs=[
            pl.BlockSpec((RT, 128), lambda i: (0, 0)),
            pl.BlockSpec((1, 128), lambda i: (0, 0)),
        ],
        out_shape=[
            jax.ShapeDtypeStruct((RT, 128), jnp.int32),
            jax.ShapeDtypeStruct((1, 128), jnp.int32),
        ],
        scratch_shapes=[pltpu.VMEM((N, E), jnp.float32)],
    )(xf, centroids)


def _make_permute(gather: bool):
    mesh = plsc.VectorSubcoreMesh(core_axis_name="c", subcore_axis_name="s")

    @functools.partial(
        pl.kernel, mesh=mesh,
        out_type=jax.ShapeDtypeStruct((N, D), jnp.float32),
        scratch_types=[
            pltpu.VMEM((SCCH, SCB), jnp.int32),
            pltpu.VMEM((SCB, D), jnp.float32),
            pltpu.SemaphoreType.DMA,
        ],
    )
    def perm(src_hbm, idx_hbm, dst_hbm, idx_v, buf, sem):
        wid = lax.axis_index("s") * 2 + lax.axis_index("c")
        base = wid * PW
        pltpu.sync_copy(idx_hbm.at[wid], idx_v)
        for j in range(SCCH):
            if gather:
                pltpu.async_copy(src_hbm.at[idx_v.at[j]], buf, sem).wait()
                pltpu.sync_copy(buf, dst_hbm.at[pl.ds(base + j * SCB, SCB)])
            else:
                pltpu.sync_copy(src_hbm.at[pl.ds(base + j * SCB, SCB)], buf)
                pltpu.async_copy(buf, dst_hbm.at[idx_v.at[j]], sem).wait()

    return perm


@functools.cache
def _permute_kernels():
    return _make_permute(gather=False), _make_permute(gather=True)


def _group_body(offs_sref, xs_ref, u_ref, v_ref, b_ref, o_ref):
    i = pl.program_id(0)
    t0 = i * GT
    xt = xs_ref[...]
    gi = t0 + lax.broadcasted_iota(jnp.int32, (GT, 1), 0)
    o_ref[...] = jnp.broadcast_to(b_ref[...], (GT, D))
    for e in range(E):
        s = offs_sref[e]
        en = offs_sref[E + e]

        @pl.when((s < t0 + GT) & (en > t0))
        def _run():
            m = ((gi >= s) & (gi < en)).astype(jnp.float32)
            xe = xt * m
            h = lax.dot_general(xe, u_ref[e], (((1,), (0,)), ((), ())),
                                preferred_element_type=jnp.float32)
            o_ref[...] += lax.dot_general(h, v_ref[e], (((1,), (0,)), ((), ())),
                                          preferred_element_type=jnp.float32)


def _group_mm(offs, xs, U, V, bias):
    return pl.pallas_call(
        _group_body,
        grid_spec=pltpu.PrefetchScalarGridSpec(
            num_scalar_prefetch=1,
            grid=(N // GT,),
            in_specs=[
                pl.BlockSpec((GT, D), lambda i, offs: (i, 0)),
                pl.BlockSpec((E, D, R), lambda i, offs: (0, 0, 0)),
                pl.BlockSpec((E, R, D), lambda i, offs: (0, 0, 0)),
                pl.BlockSpec((1, D), lambda i, offs: (0, 0)),
            ],
            out_specs=pl.BlockSpec((GT, D), lambda i, offs: (i, 0)),
        ),
        out_shape=jax.ShapeDtypeStruct((N, D), jnp.float32),
    )(offs, xs, U, V, bias.reshape(1, D))


@jax.jit
def kernel(x, centroids, U, V, bias):
    orig_shape = x.shape
    xf = x.reshape(N, D)
    dest_mat, offs_pad = _route(xf, centroids)
    dest = dest_mat[:, :NCH].T.reshape(NW, SCCH, SCB)
    offs = offs_pad[0, :2 * E]
    scatter_rows, gather_rows = _permute_kernels()
    xs = scatter_rows(xf, dest)
    ys = _group_mm(offs, xs, U, V, bias)
    out = gather_rows(ys, dest)
    return out.reshape(orig_shape)


# speedup vs baseline: 1.2608x; 1.2608x over previous
import functools

import jax
import jax.numpy as jnp
from jax import lax
from jax.experimental import pallas as pl
from jax.experimental.pallas import tpu as pltpu
from jax.experimental.pallas import tpu_sc as plsc

E = 8
D = 1024
R = 128
N = 4096
RT = 512
NCH = N // RT
GT = 256
NW = 32
PW = N // NW
GB = 32
NGB = PW // GB


def _argmin_pid(xt, c):
    xn = jnp.sum(xt * xt, axis=1, keepdims=True)
    dots = lax.dot_general(xt, c, (((1,), (1,)), ((), ())),
                           preferred_element_type=jnp.float32)
    cn = jnp.sum(c * c, axis=1)
    d2 = xn - 2.0 * dots + cn[None, :]
    bestv = d2[:, 0:1]
    bestid = jnp.zeros((xt.shape[0], 1), dtype=jnp.int32)
    for e in range(1, E):
        v = d2[:, e:e + 1]
        take = v < bestv
        bestid = jnp.where(take, e, bestid)
        bestv = jnp.where(take, v, bestv)
    return bestid


def _route_body(x_ref, c_ref, uall_ref, hsel_ref, dest_ref, offs_ref, c_s):
    i = pl.program_id(0)
    xt = x_ref[...]
    bestid = _argmin_pid(xt, c_ref[...])
    onehot = (bestid == lax.broadcasted_iota(jnp.int32, (1, E), 1))
    c_s[pl.ds(i * RT, RT), :] = onehot.astype(jnp.float32)
    h = lax.dot_general(xt, uall_ref[...], (((1,), (0,)), ((), ())),
                        preferred_element_type=jnp.float32)
    hsel = jnp.zeros((RT, R), jnp.float32)
    for e in range(E):
        m = (bestid == e).astype(jnp.float32)
        hsel = hsel + h[:, e * R:(e + 1) * R] * m
    hsel_ref[...] = hsel

    @pl.when(i == NCH - 1)
    def _finalize():
        hi = lax.Precision.HIGHEST
        ccat = jnp.concatenate(
            [c_s[ch * RT:(ch + 1) * RT, :] for ch in range(NCH)], axis=1)
        r0 = lax.broadcasted_iota(jnp.int32, (RT, RT), 0)
        r1 = lax.broadcasted_iota(jnp.int32, (RT, RT), 1)
        tlow = jnp.where(r0 >= r1, 1.0, 0.0)
        incl = lax.dot_general(tlow, ccat, (((1,), (0,)), ((), ())),
                               preferred_element_type=jnp.float32,
                               precision=hi)
        tot = incl[RT - 1:RT, :]
        li = lax.broadcasted_iota(jnp.int32, (NCH * E, NCH * E), 0)
        lj = lax.broadcasted_iota(jnp.int32, (NCH * E, NCH * E), 1)
        tch = jnp.where((li % E == lj % E) & (li // E < lj // E), 1.0, 0.0)
        prior = lax.dot_general(tot, tch, (((1,), (0,)), ((), ())),
                                preferred_element_type=jnp.float32,
                                precision=hi)
        gincl = incl + prior
        si = lax.broadcasted_iota(jnp.int32, (NCH * E, E), 0)
        sj = lax.broadcasted_iota(jnp.int32, (NCH * E, E), 1)
        counts = lax.dot_general(
            tot, jnp.where(si % E == sj, 1.0, 0.0), (((1,), (0,)), ((), ())),
            preferred_element_type=jnp.float32, precision=hi)
        bi = lax.broadcasted_iota(jnp.int32, (E, NCH * E), 0)
        bj = lax.broadcasted_iota(jnp.int32, (E, NCH * E), 1)
        starts64 = lax.dot_general(
            counts, jnp.where(bi < bj % E, 1.0, 0.0), (((1,), (0,)), ((), ())),
            preferred_element_type=jnp.float32, precision=hi)
        d64 = ccat * (starts64 + gincl - 1.0)
        gi = lax.broadcasted_iota(jnp.int32, (NCH * E, 128), 0)
        gj = lax.broadcasted_iota(jnp.int32, (NCH * E, 128), 1)
        dest = lax.dot_general(
            d64, jnp.where(gi // E == gj, 1.0, 0.0), (((1,), (0,)), ((), ())),
            preferred_element_type=jnp.float32, precision=hi)
        dest_ref[...] = dest.astype(jnp.int32)
        oi = lax.broadcasted_iota(jnp.int32, (E, 128), 0)
        oj = lax.broadcasted_iota(jnp.int32, (E, 128), 1)
        omat = jnp.where((oj < E) & (oi < oj % E), 1.0, 0.0) + \
            jnp.where((oj >= E) & (oj < 2 * E) & (oi <= oj % E), 1.0, 0.0)
        offs = lax.dot_general(counts, omat, (((1,), (0,)), ((), ())),
                               preferred_element_type=jnp.float32,
                               precision=hi)
        offs_ref[...] = offs.astype(jnp.int32)


def _route(xf, centroids, U_all):
    return pl.pallas_call(
        _route_body,
        grid=(NCH,),
        in_specs=[
            pl.BlockSpec((RT, D), lambda i: (i, 0)),
            pl.BlockSpec((E, D), lambda i: (0, 0)),
            pl.BlockSpec((D, E * R), lambda i: (0, 0)),
        ],
        out_specs=[
            pl.BlockSpec((RT, R), lambda i: (i, 0)),
            pl.BlockSpec((RT, 128), lambda i: (0, 0)),
            pl.BlockSpec((1, 128), lambda i: (0, 0)),
        ],
        out_shape=[
            jax.ShapeDtypeStruct((N, R), jnp.float32),
            jax.ShapeDtypeStruct((RT, 128), jnp.int32),
            jax.ShapeDtypeStruct((1, 128), jnp.int32),
        ],
        scratch_shapes=[pltpu.VMEM((N, E), jnp.float32)],
    )(xf, centroids, U_all)


@functools.cache
def _scatter_h_kernel():
    mesh = plsc.VectorSubcoreMesh(core_axis_name="c", subcore_axis_name="s")

    @functools.partial(
        pl.kernel, mesh=mesh,
        out_type=jax.ShapeDtypeStruct((N, R), jnp.float32),
        scratch_types=[
            pltpu.VMEM((1, PW), jnp.int32),
            pltpu.VMEM((PW, R), jnp.float32),
            pltpu.SemaphoreType.DMA,
        ],
    )
    def scat(src_hbm, idx_hbm, dst_hbm, idx_v, buf, sem):
        wid = lax.axis_index("s") * 2 + lax.axis_index("c")
        pltpu.sync_copy(idx_hbm.at[wid], idx_v)
        pltpu.sync_copy(src_hbm.at[pl.ds(wid * PW, PW)], buf)
        pltpu.async_copy(buf, dst_hbm.at[idx_v.at[0]], sem).wait()

    return scat


@functools.cache
def _gather_y_kernel():
    mesh = plsc.VectorSubcoreMesh(core_axis_name="c", subcore_axis_name="s")
    NB = 3

    @functools.partial(
        pl.kernel, mesh=mesh,
        out_type=jax.ShapeDtypeStruct((N, D), jnp.float32),
        scratch_types=[
            pltpu.VMEM((NGB, GB), jnp.int32),
            pltpu.VMEM((NB, GB, D), jnp.float32),
            pltpu.SemaphoreType.DMA,
            pltpu.SemaphoreType.DMA,
            pltpu.SemaphoreType.DMA,
        ],
    )
    def gath(src_hbm, idx_hbm, dst_hbm, idx_v, buf, sem0, sem1, sem2):
        sems = [sem0, sem1, sem2]
        wid = lax.axis_index("s") * 2 + lax.axis_index("c")
        base = wid * PW
        pltpu.sync_copy(idx_hbm.at[wid], idx_v)
        cps = []
        for j in range(NGB):
            b = j % NB
            if j >= NB:
                cps[j - NB].wait()
                pltpu.sync_copy(buf.at[(j - NB) % NB],
                                dst_hbm.at[pl.ds(base + (j - NB) * GB, GB)])
            cps.append(pltpu.async_copy(src_hbm.at[idx_v.at[j]], buf.at[b],
                                        sems[b]))
        for j in range(max(0, NGB - NB), NGB):
            cps[j].wait()
            pltpu.sync_copy(buf.at[j % NB],
                            dst_hbm.at[pl.ds(base + j * GB, GB)])

    return gath


def _group_body(offs_sref, hs_ref, v_ref, b_ref, o_ref):
    i = pl.program_id(0)
    t0 = i * GT
    ht = hs_ref[...]
    gi = t0 + lax.broadcasted_iota(jnp.int32, (GT, 1), 0)
    o_ref[...] = jnp.broadcast_to(b_ref[...], (GT, D))
    for e in range(E):
        s = offs_sref[e]
        en = offs_sref[E + e]

        @pl.when((s < t0 + GT) & (en > t0))
        def _run():
            m = ((gi >= s) & (gi < en)).astype(jnp.float32)
            he = ht * m
            o_ref[...] += lax.dot_general(he, v_ref[e], (((1,), (0,)), ((), ())),
                                          preferred_element_type=jnp.float32)


def _group_mm(offs, hs, V, bias):
    return pl.pallas_call(
        _group_body,
        grid_spec=pltpu.PrefetchScalarGridSpec(
            num_scalar_prefetch=1,
            grid=(N // GT,),
            in_specs=[
                pl.BlockSpec((GT, R), lambda i, offs: (i, 0)),
                pl.BlockSpec((E, R, D), lambda i, offs: (0, 0, 0)),
                pl.BlockSpec((1, D), lambda i, offs: (0, 0)),
            ],
            out_specs=pl.BlockSpec((GT, D), lambda i, offs: (i, 0)),
        ),
        out_shape=jax.ShapeDtypeStruct((N, D), jnp.float32),
    )(offs, hs, V, bias.reshape(1, D))


@jax.jit
def kernel(x, centroids, U, V, bias):
    orig_shape = x.shape
    xf = x.reshape(N, D)
    U_all = U.transpose(1, 0, 2).reshape(D, E * R)
    hsel, dest_mat, offs_pad = _route(xf, centroids, U_all)
    dest_flat = dest_mat[:, :NCH].T.reshape(N)
    offs = offs_pad[0, :2 * E]
    hs = _scatter_h_kernel()(hsel, dest_flat.reshape(NW, 1, PW))
    ys = _group_mm(offs, hs, V, bias)
    out = _gather_y_kernel()(ys, dest_flat.reshape(NW, NGB, GB))
    return out.reshape(orig_shape)


# speedup vs baseline: 1.2844x; 1.0188x over previous
import functools

import jax
import jax.numpy as jnp
from jax import lax
from jax.experimental import pallas as pl
from jax.experimental.pallas import tpu as pltpu
from jax.experimental.pallas import tpu_sc as plsc

E = 8
D = 1024
R = 128
N = 4096
RT = 512
NCH = N // RT
GT = 256
NW = 32
PW = N // NW
GB = 32
NGB = PW // GB


def _argmin_pid(xt, c):
    xn = jnp.sum(xt * xt, axis=1, keepdims=True)
    dots = lax.dot_general(xt, c, (((1,), (1,)), ((), ())),
                           preferred_element_type=jnp.float32)
    cn = jnp.sum(c * c, axis=1)
    d2 = xn - 2.0 * dots + cn[None, :]
    bestv = d2[:, 0:1]
    bestid = jnp.zeros((xt.shape[0], 1), dtype=jnp.int32)
    for e in range(1, E):
        v = d2[:, e:e + 1]
        take = v < bestv
        bestid = jnp.where(take, e, bestid)
        bestv = jnp.where(take, v, bestv)
    return bestid


def _route_body(x_ref, c_ref, uall_ref, hsel_ref, dest_ref, offs_ref, c_s):
    i = pl.program_id(0)
    xt = x_ref[...]
    bestid = _argmin_pid(xt, c_ref[...])
    onehot = (bestid == lax.broadcasted_iota(jnp.int32, (1, E), 1))
    c_s[pl.ds(i * RT, RT), :] = onehot.astype(jnp.float32)
    h = lax.dot_general(xt, uall_ref[...], (((1,), (0,)), ((), ())),
                        preferred_element_type=jnp.float32)
    hsel = jnp.zeros((RT, R), jnp.float32)
    for e in range(E):
        m = (bestid == e).astype(jnp.float32)
        hsel = hsel + h[:, e * R:(e + 1) * R] * m
    hsel_ref[...] = hsel

    @pl.when(i == NCH - 1)
    def _finalize():
        hi = lax.Precision.HIGHEST
        ccat = jnp.concatenate(
            [c_s[ch * RT:(ch + 1) * RT, :] for ch in range(NCH)], axis=1)
        r0 = lax.broadcasted_iota(jnp.int32, (RT, RT), 0)
        r1 = lax.broadcasted_iota(jnp.int32, (RT, RT), 1)
        tlow = jnp.where(r0 >= r1, 1.0, 0.0)
        incl = lax.dot_general(tlow, ccat, (((1,), (0,)), ((), ())),
                               preferred_element_type=jnp.float32)
        tot = incl[RT - 1:RT, :]
        li = lax.broadcasted_iota(jnp.int32, (NCH * E, NCH * E), 0)
        lj = lax.broadcasted_iota(jnp.int32, (NCH * E, NCH * E), 1)
        tch = jnp.where((li % E == lj % E) & (li // E < lj // E), 1.0, 0.0)
        prior = lax.dot_general(tot, tch, (((1,), (0,)), ((), ())),
                                preferred_element_type=jnp.float32,
                                precision=hi)
        gincl = incl + prior
        si = lax.broadcasted_iota(jnp.int32, (NCH * E, E), 0)
        sj = lax.broadcasted_iota(jnp.int32, (NCH * E, E), 1)
        counts = lax.dot_general(
            tot, jnp.where(si % E == sj, 1.0, 0.0), (((1,), (0,)), ((), ())),
            preferred_element_type=jnp.float32, precision=hi)
        bi = lax.broadcasted_iota(jnp.int32, (E, NCH * E), 0)
        bj = lax.broadcasted_iota(jnp.int32, (E, NCH * E), 1)
        starts64 = lax.dot_general(
            counts, jnp.where(bi < bj % E, 1.0, 0.0), (((1,), (0,)), ((), ())),
            preferred_element_type=jnp.float32, precision=hi)
        d64 = ccat * (starts64 + gincl - 1.0)
        gi = lax.broadcasted_iota(jnp.int32, (NCH * E, 128), 0)
        gj = lax.broadcasted_iota(jnp.int32, (NCH * E, 128), 1)
        dest = lax.dot_general(
            d64, jnp.where(gi // E == gj, 1.0, 0.0), (((1,), (0,)), ((), ())),
            preferred_element_type=jnp.float32, precision=hi)
        dest_ref[...] = dest.astype(jnp.int32)
        oi = lax.broadcasted_iota(jnp.int32, (E, 128), 0)
        oj = lax.broadcasted_iota(jnp.int32, (E, 128), 1)
        omat = jnp.where((oj < E) & (oi < oj % E), 1.0, 0.0) + \
            jnp.where((oj >= E) & (oj < 2 * E) & (oi <= oj % E), 1.0, 0.0)
        offs = lax.dot_general(counts, omat, (((1,), (0,)), ((), ())),
                               preferred_element_type=jnp.float32,
                               precision=hi)
        offs_ref[...] = offs.astype(jnp.int32)


def _route(xf, centroids, U_all):
    return pl.pallas_call(
        _route_body,
        grid=(NCH,),
        in_specs=[
            pl.BlockSpec((RT, D), lambda i: (i, 0)),
            pl.BlockSpec((E, D), lambda i: (0, 0)),
            pl.BlockSpec((D, E * R), lambda i: (0, 0)),
        ],
        out_specs=[
            pl.BlockSpec((RT, R), lambda i: (i, 0)),
            pl.BlockSpec((RT, 128), lambda i: (0, 0)),
            pl.BlockSpec((1, 128), lambda i: (0, 0)),
        ],
        out_shape=[
            jax.ShapeDtypeStruct((N, R), jnp.float32),
            jax.ShapeDtypeStruct((RT, 128), jnp.int32),
            jax.ShapeDtypeStruct((1, 128), jnp.int32),
        ],
        scratch_shapes=[pltpu.VMEM((N, E), jnp.float32)],
    )(xf, centroids, U_all)


@functools.cache
def _scatter_h_kernel():
    mesh = plsc.VectorSubcoreMesh(core_axis_name="c", subcore_axis_name="s")

    @functools.partial(
        pl.kernel, mesh=mesh,
        out_type=jax.ShapeDtypeStruct((N, R), jnp.float32),
        scratch_types=[
            pltpu.VMEM((1, PW), jnp.int32),
            pltpu.VMEM((PW, R), jnp.float32),
            pltpu.SemaphoreType.DMA,
        ],
    )
    def scat(src_hbm, idx_hbm, dst_hbm, idx_v, buf, sem):
        wid = lax.axis_index("s") * 2 + lax.axis_index("c")
        pltpu.sync_copy(idx_hbm.at[wid], idx_v)
        pltpu.sync_copy(src_hbm.at[pl.ds(wid * PW, PW)], buf)
        pltpu.async_copy(buf, dst_hbm.at[idx_v.at[0]], sem).wait()

    return scat


@functools.cache
def _gather_y_kernel():
    mesh = plsc.VectorSubcoreMesh(core_axis_name="c", subcore_axis_name="s")
    NB = 3

    @functools.partial(
        pl.kernel, mesh=mesh,
        out_type=jax.ShapeDtypeStruct((N, D), jnp.float32),
        scratch_types=[
            pltpu.VMEM((NGB, GB), jnp.int32),
            pltpu.VMEM((NB, GB, D), jnp.float32),
            pltpu.SemaphoreType.DMA,
            pltpu.SemaphoreType.DMA,
            pltpu.SemaphoreType.DMA,
        ],
    )
    def gath(src_hbm, idx_hbm, dst_hbm, idx_v, buf, sem0, sem1, sem2):
        sems = [sem0, sem1, sem2]
        wid = lax.axis_index("s") * 2 + lax.axis_index("c")
        base = wid * PW
        pltpu.sync_copy(idx_hbm.at[wid], idx_v)
        cps = []
        for j in range(NGB):
            b = j % NB
            if j >= NB:
                cps[j - NB].wait()
                pltpu.sync_copy(buf.at[(j - NB) % NB],
                                dst_hbm.at[pl.ds(base + (j - NB) * GB, GB)])
            cps.append(pltpu.async_copy(src_hbm.at[idx_v.at[j]], buf.at[b],
                                        sems[b]))
        for j in range(max(0, NGB - NB), NGB):
            cps[j].wait()
            pltpu.sync_copy(buf.at[j % NB],
                            dst_hbm.at[pl.ds(base + j * GB, GB)])

    return gath


def _group_body(offs_sref, hs_ref, v_ref, b_ref, o_ref):
    i = pl.program_id(0)
    t0 = i * GT
    ht = hs_ref[...]
    gi = t0 + lax.broadcasted_iota(jnp.int32, (GT, 1), 0)
    e0 = jnp.int32(0)
    e1 = jnp.int32(0)
    for e in range(1, E):
        e0 = jnp.where(offs_sref[e] <= t0, e, e0)
        e1 = jnp.where(offs_sref[e] <= t0 + GT - 1, e, e1)
    s0 = offs_sref[e0]
    en0 = offs_sref[E + e0]
    m0 = ((gi >= s0) & (gi < en0)).astype(jnp.float32)
    e0p = jnp.minimum(e0 + 1, E - 1)
    s1 = offs_sref[e0p]
    en1 = offs_sref[E + e0p]
    m1 = ((gi >= s1) & (gi < en1) & (e0p > e0)).astype(jnp.float32)
    y = jnp.broadcast_to(b_ref[...], (GT, D))
    y = y + lax.dot_general(ht * m0, v_ref[e0], (((1,), (0,)), ((), ())),
                            preferred_element_type=jnp.float32)
    y = y + lax.dot_general(ht * m1, v_ref[e0p], (((1,), (0,)), ((), ())),
                            preferred_element_type=jnp.float32)
    o_ref[...] = y
    for k in range(2, E):
        @pl.when(e0 + k <= e1)
        def _extra():
            ek = jnp.minimum(e0 + k, E - 1)
            sk = offs_sref[ek]
            enk = offs_sref[E + ek]
            mk = ((gi >= sk) & (gi < enk)).astype(jnp.float32)
            o_ref[...] += lax.dot_general(
                ht * mk, v_ref[ek], (((1,), (0,)), ((), ())),
                preferred_element_type=jnp.float32)


def _group_mm(offs, hs, V, bias):
    return pl.pallas_call(
        _group_body,
        grid_spec=pltpu.PrefetchScalarGridSpec(
            num_scalar_prefetch=1,
            grid=(N // GT,),
            in_specs=[
                pl.BlockSpec((GT, R), lambda i, offs: (i, 0)),
                pl.BlockSpec((E, R, D), lambda i, offs: (0, 0, 0)),
                pl.BlockSpec((1, D), lambda i, offs: (0, 0)),
            ],
            out_specs=pl.BlockSpec((GT, D), lambda i, offs: (i, 0)),
        ),
        out_shape=jax.ShapeDtypeStruct((N, D), jnp.float32),
    )(offs, hs, V, bias.reshape(1, D))


@jax.jit
def kernel(x, centroids, U, V, bias):
    orig_shape = x.shape
    xf = x.reshape(N, D)
    U_all = U.transpose(1, 0, 2).reshape(D, E * R)
    hsel, dest_mat, offs_pad = _route(xf, centroids, U_all)
    dest_flat = dest_mat[:, :NCH].T.reshape(N)
    offs = offs_pad[0, :2 * E]
    hs = _scatter_h_kernel()(hsel, dest_flat.reshape(NW, 1, PW))
    ys = _group_mm(offs, hs, V, bias)
    out = _gather_y_kernel()(ys, dest_flat.reshape(NW, NGB, GB))
    return out.reshape(orig_shape)


# speedup vs baseline: 1.3407x; 1.0438x over previous
import functools

import jax
import jax.numpy as jnp
from jax import lax
from jax.experimental import pallas as pl
from jax.experimental.pallas import tpu as pltpu
from jax.experimental.pallas import tpu_sc as plsc

E = 8
D = 1024
R = 128
N = 4096
RT = 512
NCH = N // RT
GT = 512
NW = 32
PW = N // NW
GB = 32
NGB = PW // GB


def _argmin_pid(xt, c):
    xn = jnp.sum(xt * xt, axis=1, keepdims=True)
    dots = lax.dot_general(xt, c, (((1,), (1,)), ((), ())),
                           preferred_element_type=jnp.float32)
    cn = jnp.sum(c * c, axis=1)
    d2 = xn - 2.0 * dots + cn[None, :]
    bestv = d2[:, 0:1]
    bestid = jnp.zeros((xt.shape[0], 1), dtype=jnp.int32)
    for e in range(1, E):
        v = d2[:, e:e + 1]
        take = v < bestv
        bestid = jnp.where(take, e, bestid)
        bestv = jnp.where(take, v, bestv)
    return bestid


def _route_body(x_ref, c_ref, uall_ref, hsel_ref, dest_ref, offs_ref, c_s):
    i = pl.program_id(0)
    xt = x_ref[...]
    bestid = _argmin_pid(xt, c_ref[...])
    onehot = (bestid == lax.broadcasted_iota(jnp.int32, (1, E), 1))
    c_s[pl.ds(i * RT, RT), :] = onehot.astype(jnp.float32)
    h = lax.dot_general(xt, uall_ref[...], (((1,), (0,)), ((), ())),
                        preferred_element_type=jnp.float32)
    hsel = jnp.zeros((RT, R), jnp.float32)
    for e in range(E):
        m = (bestid == e).astype(jnp.float32)
        hsel = hsel + h[:, e * R:(e + 1) * R] * m
    hsel_ref[...] = hsel

    @pl.when(i == NCH - 1)
    def _finalize():
        hi = lax.Precision.HIGHEST
        ccat = jnp.concatenate(
            [c_s[ch * RT:(ch + 1) * RT, :] for ch in range(NCH)], axis=1)
        r0 = lax.broadcasted_iota(jnp.int32, (RT, RT), 0)
        r1 = lax.broadcasted_iota(jnp.int32, (RT, RT), 1)
        tlow = jnp.where(r0 >= r1, 1.0, 0.0)
        incl = lax.dot_general(tlow, ccat, (((1,), (0,)), ((), ())),
                               preferred_element_type=jnp.float32)
        tot = incl[RT - 1:RT, :]
        li = lax.broadcasted_iota(jnp.int32, (NCH * E, NCH * E), 0)
        lj = lax.broadcasted_iota(jnp.int32, (NCH * E, NCH * E), 1)
        tch = jnp.where((li % E == lj % E) & (li // E < lj // E), 1.0, 0.0)
        prior = lax.dot_general(tot, tch, (((1,), (0,)), ((), ())),
                                preferred_element_type=jnp.float32,
                                precision=hi)
        gincl = incl + prior
        si = lax.broadcasted_iota(jnp.int32, (NCH * E, E), 0)
        sj = lax.broadcasted_iota(jnp.int32, (NCH * E, E), 1)
        counts = lax.dot_general(
            tot, jnp.where(si % E == sj, 1.0, 0.0), (((1,), (0,)), ((), ())),
            preferred_element_type=jnp.float32, precision=hi)
        bi = lax.broadcasted_iota(jnp.int32, (E, NCH * E), 0)
        bj = lax.broadcasted_iota(jnp.int32, (E, NCH * E), 1)
        starts64 = lax.dot_general(
            counts, jnp.where(bi < bj % E, 1.0, 0.0), (((1,), (0,)), ((), ())),
            preferred_element_type=jnp.float32, precision=hi)
        d64 = ccat * (starts64 + gincl - 1.0)
        gi = lax.broadcasted_iota(jnp.int32, (NCH * E, 128), 0)
        gj = lax.broadcasted_iota(jnp.int32, (NCH * E, 128), 1)
        dest = lax.dot_general(
            d64, jnp.where(gi // E == gj, 1.0, 0.0), (((1,), (0,)), ((), ())),
            preferred_element_type=jnp.float32, precision=hi)
        dest_ref[...] = dest.astype(jnp.int32)
        oi = lax.broadcasted_iota(jnp.int32, (E, 128), 0)
        oj = lax.broadcasted_iota(jnp.int32, (E, 128), 1)
        omat = jnp.where((oj < E) & (oi < oj % E), 1.0, 0.0) + \
            jnp.where((oj >= E) & (oj < 2 * E) & (oi <= oj % E), 1.0, 0.0)
        offs = lax.dot_general(counts, omat, (((1,), (0,)), ((), ())),
                               preferred_element_type=jnp.float32,
                               precision=hi)
        offs_ref[...] = offs.astype(jnp.int32)


def _route(xf, centroids, U_all):
    return pl.pallas_call(
        _route_body,
        grid=(NCH,),
        in_specs=[
            pl.BlockSpec((RT, D), lambda i: (i, 0)),
            pl.BlockSpec((E, D), lambda i: (0, 0)),
            pl.BlockSpec((D, E * R), lambda i: (0, 0)),
        ],
        out_specs=[
            pl.BlockSpec((RT, R), lambda i: (i, 0)),
            pl.BlockSpec((RT, 128), lambda i: (0, 0)),
            pl.BlockSpec((1, 128), lambda i: (0, 0)),
        ],
        out_shape=[
            jax.ShapeDtypeStruct((N, R), jnp.float32),
            jax.ShapeDtypeStruct((RT, 128), jnp.int32),
            jax.ShapeDtypeStruct((1, 128), jnp.int32),
        ],
        scratch_shapes=[pltpu.VMEM((N, E), jnp.float32)],
    )(xf, centroids, U_all)


@functools.cache
def _scatter_h_kernel():
    mesh = plsc.VectorSubcoreMesh(core_axis_name="c", subcore_axis_name="s")

    @functools.partial(
        pl.kernel, mesh=mesh,
        out_type=jax.ShapeDtypeStruct((N, R), jnp.float32),
        scratch_types=[
            pltpu.VMEM((1, PW), jnp.int32),
            pltpu.VMEM((PW, R), jnp.float32),
            pltpu.SemaphoreType.DMA,
        ],
    )
    def scat(src_hbm, idx_hbm, dst_hbm, idx_v, buf, sem):
        wid = lax.axis_index("s") * 2 + lax.axis_index("c")
        pltpu.sync_copy(idx_hbm.at[wid], idx_v)
        pltpu.sync_copy(src_hbm.at[pl.ds(wid * PW, PW)], buf)
        pltpu.async_copy(buf, dst_hbm.at[idx_v.at[0]], sem).wait()

    return scat


@functools.cache
def _gather_y_kernel():
    mesh = plsc.VectorSubcoreMesh(core_axis_name="c", subcore_axis_name="s")
    NB = 3

    @functools.partial(
        pl.kernel, mesh=mesh,
        out_type=jax.ShapeDtypeStruct((N, D), jnp.float32),
        scratch_types=[
            pltpu.VMEM((NGB, GB), jnp.int32),
            pltpu.VMEM((NB, GB, D), jnp.float32),
            pltpu.SemaphoreType.DMA,
            pltpu.SemaphoreType.DMA,
            pltpu.SemaphoreType.DMA,
        ],
    )
    def gath(src_hbm, idx_hbm, dst_hbm, idx_v, buf, sem0, sem1, sem2):
        sems = [sem0, sem1, sem2]
        wid = lax.axis_index("s") * 2 + lax.axis_index("c")
        base = wid * PW
        pltpu.sync_copy(idx_hbm.at[wid], idx_v)
        cps = []
        for j in range(NGB):
            b = j % NB
            if j >= NB:
                cps[j - NB].wait()
                pltpu.sync_copy(buf.at[(j - NB) % NB],
                                dst_hbm.at[pl.ds(base + (j - NB) * GB, GB)])
            cps.append(pltpu.async_copy(src_hbm.at[idx_v.at[j]], buf.at[b],
                                        sems[b]))
        for j in range(max(0, NGB - NB), NGB):
            cps[j].wait()
            pltpu.sync_copy(buf.at[j % NB],
                            dst_hbm.at[pl.ds(base + j * GB, GB)])

    return gath


def _group_body(offs_sref, hs_ref, v_ref, b_ref, o_ref):
    i = pl.program_id(0)
    t0 = i * GT
    ht = hs_ref[...]
    gi = t0 + lax.broadcasted_iota(jnp.int32, (GT, 1), 0)
    e0 = jnp.int32(0)
    e1 = jnp.int32(0)
    for e in range(1, E):
        e0 = jnp.where(offs_sref[e] <= t0, e, e0)
        e1 = jnp.where(offs_sref[e] <= t0 + GT - 1, e, e1)
    s0 = offs_sref[e0]
    en0 = offs_sref[E + e0]
    m0 = ((gi >= s0) & (gi < en0)).astype(jnp.float32)
    e0p = jnp.minimum(e0 + 1, E - 1)
    s1 = offs_sref[e0p]
    en1 = offs_sref[E + e0p]
    m1 = ((gi >= s1) & (gi < en1) & (e0p > e0)).astype(jnp.float32)
    y = jnp.broadcast_to(b_ref[...], (GT, D))
    y = y + lax.dot_general(ht * m0, v_ref[e0], (((1,), (0,)), ((), ())),
                            preferred_element_type=jnp.float32)
    y = y + lax.dot_general(ht * m1, v_ref[e0p], (((1,), (0,)), ((), ())),
                            preferred_element_type=jnp.float32)
    o_ref[...] = y
    for k in range(2, E):
        @pl.when(e0 + k <= e1)
        def _extra():
            ek = jnp.minimum(e0 + k, E - 1)
            sk = offs_sref[ek]
            enk = offs_sref[E + ek]
            mk = ((gi >= sk) & (gi < enk)).astype(jnp.float32)
            o_ref[...] += lax.dot_general(
                ht * mk, v_ref[ek], (((1,), (0,)), ((), ())),
                preferred_element_type=jnp.float32)


def _group_mm(offs, hs, V, bias):
    return pl.pallas_call(
        _group_body,
        grid_spec=pltpu.PrefetchScalarGridSpec(
            num_scalar_prefetch=1,
            grid=(N // GT,),
            in_specs=[
                pl.BlockSpec((GT, R), lambda i, offs: (i, 0)),
                pl.BlockSpec((E, R, D), lambda i, offs: (0, 0, 0)),
                pl.BlockSpec((1, D), lambda i, offs: (0, 0)),
            ],
            out_specs=pl.BlockSpec((GT, D), lambda i, offs: (i, 0)),
        ),
        out_shape=jax.ShapeDtypeStruct((N, D), jnp.float32),
    )(offs, hs, V, bias.reshape(1, D))


@jax.jit
def kernel(x, centroids, U, V, bias):
    orig_shape = x.shape
    xf = x.reshape(N, D)
    U_all = U.transpose(1, 0, 2).reshape(D, E * R)
    hsel, dest_mat, offs_pad = _route(xf, centroids, U_all)
    dest_flat = dest_mat[:, :NCH].T.reshape(N)
    offs = offs_pad[0, :2 * E]
    hs = _scatter_h_kernel()(hsel, dest_flat.reshape(NW, 1, PW))
    ys = _group_mm(offs, hs, V, bias)
    out = _gather_y_kernel()(ys, dest_flat.reshape(NW, NGB, GB))
    return out.reshape(orig_shape)
